# Initial kernel scaffold; baseline (speedup 1.0000x reference)
import jax
import jax.numpy as jnp
from jax.experimental import pallas as pl


def kernel(x_cat, tables):
    raise NotImplementedError("write your pallas kernel here")


# speedup vs baseline: 1.1525x; 1.1525x over previous
import functools

import jax
import jax.numpy as jnp
from jax import lax
from jax.experimental import pallas as pl
from jax.experimental.pallas import tpu as pltpu
from jax.experimental.pallas import tpu_sc as plsc

NUM_FIELDS = 26
VOCAB = 100000
EMBED_DIM = 16
BATCH = 16384

N = BATCH * NUM_FIELDS
NC = 2
NS = 16
NW = NC * NS
PER_W = N // NW
CH = 1664
NCH = PER_W // CH
SUB = 128
NSUB = CH // SUB
LANES = 16
OFF_PERIOD = 208


def _body(tab_hbm, xf_hbm, out_hbm, offpat_v, idx_v, rows_v, g_sem):
    cid = lax.axis_index("c")
    sid = lax.axis_index("s")
    wid = sid * NC + cid
    base = wid * PER_W

    for k in range(OFF_PERIOD // LANES):
        off = ((lax.iota(jnp.int32, LANES) + k * LANES) % NUM_FIELDS) * VOCAB
        offpat_v[pl.ds(k * LANES, LANES)] = off

    def load_and_offset(c, buf):
        cb = base + c * CH
        pltpu.sync_copy(xf_hbm.at[pl.ds(cb, CH)], idx_v.at[buf])

        def jbody(j, _):
            sl = pl.ds(j * LANES, LANES)
            po = pl.ds((j % (OFF_PERIOD // LANES)) * LANES, LANES)
            idx_v[buf, sl] = idx_v[buf, sl] + offpat_v[po]
            return 0

        lax.fori_loop(0, CH // LANES, jbody, 0, unroll=8)

    def fire_gathers(buf):
        return [
            pltpu.async_copy(
                tab_hbm.at[idx_v.at[buf, pl.ds(s * SUB, SUB)]],
                rows_v.at[buf, pl.ds(s * SUB, SUB)],
                g_sem,
            )
            for s in range(NSUB)
        ]

    def drain_gathers(descs):
        for d in descs:
            d.wait()

    def store(c, buf):
        cb = base + c * CH
        pltpu.sync_copy(rows_v.at[buf], out_hbm.at[pl.ds(cb, CH)])

    load_and_offset(0, 0)
    descs = fire_gathers(0)
    for c in range(1, NCH):
        buf, pbuf = c % 2, (c - 1) % 2
        load_and_offset(c, buf)
        drain_gathers(descs)
        descs = fire_gathers(buf)
        store(c - 1, pbuf)
    drain_gathers(descs)
    store(NCH - 1, (NCH - 1) % 2)


@jax.jit
def _run(tab_flat, x_flat):
    mesh = plsc.VectorSubcoreMesh(core_axis_name="c", subcore_axis_name="s")
    return pl.kernel(
        _body,
        out_type=jax.ShapeDtypeStruct((N, EMBED_DIM), jnp.float32),
        mesh=mesh,
        scratch_types=[
            pltpu.VMEM((OFF_PERIOD,), jnp.int32),
            pltpu.VMEM((2, CH), jnp.int32),
            pltpu.VMEM((2, CH, EMBED_DIM), jnp.float32),
            pltpu.SemaphoreType.DMA,
        ],
        compiler_params=pltpu.CompilerParams(use_tc_tiling_on_sc=False),
    )(tab_flat, x_flat)


def kernel(x_cat, tables):
    tab_flat = tables.reshape(NUM_FIELDS * VOCAB, EMBED_DIM)
    x_flat = x_cat.reshape(-1).astype(jnp.int32)
    out = _run(tab_flat, x_flat)
    return out.reshape(BATCH, NUM_FIELDS * EMBED_DIM)


# speedup vs baseline: 6.1199x; 5.3102x over previous
import functools

import jax
import jax.numpy as jnp
from jax import lax
from jax.experimental import pallas as pl
from jax.experimental.pallas import tpu as pltpu
from jax.experimental.pallas import tpu_sc as plsc

NUM_FIELDS = 26
VOCAB = 100000
EMBED_DIM = 16
BATCH = 16384

NPAIR = NUM_FIELDS * EMBED_DIM
NC = 2
NS = 16
NW = NC * NS
PER_W = NPAIR // NW
QTR = BATCH // 4
LANES = 16


def _body(tab_hbm, xc_hbm, out_hbm, row_v, idx_v, out_v, row_sem, idx_sem, out_sem):
    cid = lax.axis_index("c")
    sid = lax.axis_index("s")
    wid = sid * NC + cid

    def gather_blk(buf):
        def blk(k, _):
            sl = pl.ds(buf * QTR + k * LANES, LANES)
            out_v[sl] = plsc.load_gather(row_v, [idx_v[sl]])
            return 0

        lax.fori_loop(0, QTR // LANES, blk, 0, unroll=8)

    def start_idx(f, q):
        return pltpu.async_copy(
            xc_hbm.at[f, pl.ds(q * QTR, QTR)],
            idx_v.at[pl.ds((q % 2) * QTR, QTR)],
            idx_sem,
        )

    def start_out(p, q):
        return pltpu.async_copy(
            out_v.at[pl.ds((q % 2) * QTR, QTR)],
            out_hbm.at[p, pl.ds(q * QTR, QTR)],
            out_sem,
        )

    for i in range(PER_W):
        p = wid * PER_W + i
        f = p // EMBED_DIM
        d = p % EMBED_DIM

        row_cp = pltpu.async_copy(tab_hbm.at[f, d, :], row_v, row_sem)
        idx_cps = [start_idx(f, 0), start_idx(f, 1)]
        out_cps = [None, None, None, None]
        row_cp.wait()
        for q in range(4):
            idx_cps[q].wait()
            if q >= 2:
                out_cps[q - 2].wait()
            gather_blk(q % 2)
            if q < 2:
                idx_cps.append(start_idx(f, q + 2))
            out_cps[q] = start_out(p, q)
        out_cps[2].wait()
        out_cps[3].wait()


@jax.jit
def _run(tab_t, xc_t):
    mesh = plsc.VectorSubcoreMesh(core_axis_name="c", subcore_axis_name="s")
    return pl.kernel(
        _body,
        out_type=jax.ShapeDtypeStruct((NPAIR, BATCH), jnp.float32),
        mesh=mesh,
        scratch_types=[
            pltpu.VMEM((VOCAB,), jnp.float32),
            pltpu.VMEM((2 * QTR,), jnp.int32),
            pltpu.VMEM((2 * QTR,), jnp.float32),
            pltpu.SemaphoreType.DMA,
            pltpu.SemaphoreType.DMA,
            pltpu.SemaphoreType.DMA,
        ],
        compiler_params=pltpu.CompilerParams(
            use_tc_tiling_on_sc=True, needs_layout_passes=False
        ),
    )(tab_t, xc_t)


def kernel(x_cat, tables):
    tab_t = jnp.transpose(tables, (0, 2, 1))
    xc_t = jnp.transpose(x_cat.astype(jnp.int32))
    out_t = _run(tab_t, xc_t)
    return jnp.transpose(out_t)


# speedup vs baseline: 10.5722x; 1.7275x over previous
import functools

import jax
import jax.numpy as jnp
from jax import lax
from jax.experimental import pallas as pl
from jax.experimental.pallas import tpu as pltpu
from jax.experimental.pallas import tpu_sc as plsc

NUM_FIELDS = 26
VOCAB = 100000
EMBED_DIM = 16
BATCH = 16384

NPAIR = NUM_FIELDS * EMBED_DIM
NC = 2
NS = 16
NW = NC * NS
PER_W = NPAIR // NW
QTR = BATCH // 4
LANES = 16


def _body(tab_hbm, xc_hbm, out_hbm, row_v, idx_v, out_v, row_sem, idx_sem, out_sem):
    cid = lax.axis_index("c")
    sid = lax.axis_index("s")
    wid = sid * NC + cid

    def gather_blk(buf):
        @plsc.parallel_loop(buf * QTR, (buf + 1) * QTR, LANES, unroll=8)
        def _blk(off):
            sl = pl.ds(off, LANES)
            out_v[sl] = plsc.load_gather(row_v, [idx_v[sl]])

    def start_idx(f, q):
        return pltpu.async_copy(
            xc_hbm.at[f, pl.ds(q * QTR, QTR)],
            idx_v.at[pl.ds((q % 2) * QTR, QTR)],
            idx_sem,
        )

    def start_out(p, q):
        return pltpu.async_copy(
            out_v.at[pl.ds((q % 2) * QTR, QTR)],
            out_hbm.at[p, pl.ds(q * QTR, QTR)],
            out_sem,
        )

    for i in range(PER_W):
        p = wid * PER_W + i
        f = p // EMBED_DIM
        d = p % EMBED_DIM

        row_cp = pltpu.async_copy(tab_hbm.at[f, d, :], row_v, row_sem)
        idx_cps = [start_idx(f, 0), start_idx(f, 1)]
        out_cps = [None, None, None, None]
        row_cp.wait()
        for q in range(4):
            idx_cps[q].wait()
            if q >= 2:
                out_cps[q - 2].wait()
            gather_blk(q % 2)
            if q < 2:
                idx_cps.append(start_idx(f, q + 2))
            out_cps[q] = start_out(p, q)
        out_cps[2].wait()
        out_cps[3].wait()


@jax.jit
def _run(tab_t, xc_t):
    mesh = plsc.VectorSubcoreMesh(core_axis_name="c", subcore_axis_name="s")
    return pl.kernel(
        _body,
        out_type=jax.ShapeDtypeStruct((NPAIR, BATCH), jnp.float32),
        mesh=mesh,
        scratch_types=[
            pltpu.VMEM((VOCAB,), jnp.float32),
            pltpu.VMEM((2 * QTR,), jnp.int32),
            pltpu.VMEM((2 * QTR,), jnp.float32),
            pltpu.SemaphoreType.DMA,
            pltpu.SemaphoreType.DMA,
            pltpu.SemaphoreType.DMA,
        ],
        compiler_params=pltpu.CompilerParams(
            use_tc_tiling_on_sc=True, needs_layout_passes=False
        ),
    )(tab_t, xc_t)


def kernel(x_cat, tables):
    tab_t = jnp.transpose(tables, (0, 2, 1))
    xc_t = jnp.transpose(x_cat.astype(jnp.int32))
    out_t = _run(tab_t, xc_t)
    return jnp.transpose(out_t)


# speedup vs baseline: 11.4660x; 1.0845x over previous
import functools

import jax
import jax.numpy as jnp
from jax import lax
from jax.experimental import pallas as pl
from jax.experimental.pallas import tpu as pltpu
from jax.experimental.pallas import tpu_sc as plsc

NUM_FIELDS = 26
VOCAB = 100000
EMBED_DIM = 16
BATCH = 16384

NPAIR = NUM_FIELDS * EMBED_DIM
NC = 2
NS = 16
NW = NC * NS
PER_W = NPAIR // NW
QTR = BATCH // 4
LANES = 16


def _body(tab_hbm, xc_hbm, out_hbm, row_v, idx_v, out_v, row_sem, idx_sem, out_sem):
    cid = lax.axis_index("c")
    sid = lax.axis_index("s")
    wid = sid * NC + cid

    def gather_quarter(q):
        src = q * QTR
        dst = (q % 2) * QTR

        @plsc.parallel_loop(0, QTR, LANES, unroll=8)
        def _blk(off):
            out_v[pl.ds(dst + off, LANES)] = plsc.load_gather(
                row_v, [idx_v[pl.ds(src + off, LANES)]]
            )

    def start_out(p, q):
        return pltpu.async_copy(
            out_v.at[pl.ds((q % 2) * QTR, QTR)],
            out_hbm.at[p, pl.ds(q * QTR, QTR)],
            out_sem,
        )

    f_prev = jnp.int32(-1)
    for i in range(PER_W):
        p = wid * PER_W + i
        f = p // EMBED_DIM
        d = p % EMBED_DIM

        row_cp = pltpu.async_copy(tab_hbm.at[f, d, :], row_v, row_sem)

        @pl.when(f != f_prev)
        def _load_idx():
            pltpu.async_copy(xc_hbm.at[f, :], idx_v, idx_sem).wait()

        f_prev = f
        row_cp.wait()
        out_cps = [None, None, None, None]
        for q in range(4):
            if q >= 2:
                out_cps[q - 2].wait()
            gather_quarter(q)
            out_cps[q] = start_out(p, q)
        out_cps[2].wait()
        out_cps[3].wait()


@jax.jit
def _run(tab_t, xc_t):
    mesh = plsc.VectorSubcoreMesh(core_axis_name="c", subcore_axis_name="s")
    return pl.kernel(
        _body,
        out_type=jax.ShapeDtypeStruct((NPAIR, BATCH), jnp.float32),
        mesh=mesh,
        scratch_types=[
            pltpu.VMEM((VOCAB,), jnp.float32),
            pltpu.VMEM((BATCH,), jnp.int32),
            pltpu.VMEM((2 * QTR,), jnp.float32),
            pltpu.SemaphoreType.DMA,
            pltpu.SemaphoreType.DMA,
            pltpu.SemaphoreType.DMA,
        ],
        compiler_params=pltpu.CompilerParams(
            use_tc_tiling_on_sc=True, needs_layout_passes=False
        ),
    )(tab_t, xc_t)


def kernel(x_cat, tables):
    tab_t = jnp.transpose(tables, (0, 2, 1))
    xc_t = jnp.transpose(x_cat.astype(jnp.int32))
    out_t = _run(tab_t, xc_t)
    return jnp.transpose(out_t)


# speedup vs baseline: 11.5440x; 1.0068x over previous
import functools

import jax
import jax.numpy as jnp
from jax import lax
from jax.experimental import pallas as pl
from jax.experimental.pallas import tpu as pltpu
from jax.experimental.pallas import tpu_sc as plsc

NUM_FIELDS = 26
VOCAB = 100000
EMBED_DIM = 16
BATCH = 16384

NPAIR = NUM_FIELDS * EMBED_DIM
NC = 2
NS = 16
NW = NC * NS
PER_W = NPAIR // NW
QTR = BATCH // 4
LANES = 16


def _body(tab_hbm, xc_hbm, out_hbm, row_v, idx_v, out_v, row_sem, idx_sem, out_sem):
    cid = lax.axis_index("c")
    sid = lax.axis_index("s")
    wid = sid * NC + cid

    def gather_quarter(q):
        src = q * QTR
        dst = (q % 2) * QTR

        @plsc.parallel_loop(0, QTR, LANES, unroll=8)
        def _blk(off):
            out_v[pl.ds(dst + off, LANES)] = plsc.load_gather(
                row_v, [idx_v[pl.ds(src + off, LANES)]]
            )

    def start_out(p, q):
        return pltpu.async_copy(
            out_v.at[pl.ds((q % 2) * QTR, QTR)],
            out_hbm.at[p, pl.ds(q * QTR, QTR)],
            out_sem,
        )

    f_prev = jnp.int32(-1)
    for j in range(PER_W):
        i = (j + sid) % PER_W
        p = wid * PER_W + i
        f = p // EMBED_DIM
        d = p % EMBED_DIM

        row_cp = pltpu.async_copy(tab_hbm.at[f, d, :], row_v, row_sem)

        @pl.when(f != f_prev)
        def _load_idx():
            pltpu.async_copy(xc_hbm.at[f, :], idx_v, idx_sem).wait()

        f_prev = f
        row_cp.wait()
        out_cps = [None, None, None, None]
        for q in range(4):
            if q >= 2:
                out_cps[q - 2].wait()
            gather_quarter(q)
            out_cps[q] = start_out(p, q)
        out_cps[2].wait()
        out_cps[3].wait()


@jax.jit
def _run(tab_t, xc_t):
    mesh = plsc.VectorSubcoreMesh(core_axis_name="c", subcore_axis_name="s")
    return pl.kernel(
        _body,
        out_type=jax.ShapeDtypeStruct((NPAIR, BATCH), jnp.float32),
        mesh=mesh,
        scratch_types=[
            pltpu.VMEM((VOCAB,), jnp.float32),
            pltpu.VMEM((BATCH,), jnp.int32),
            pltpu.VMEM((2 * QTR,), jnp.float32),
            pltpu.SemaphoreType.DMA,
            pltpu.SemaphoreType.DMA,
            pltpu.SemaphoreType.DMA,
        ],
        compiler_params=pltpu.CompilerParams(
            use_tc_tiling_on_sc=True, needs_layout_passes=False
        ),
    )(tab_t, xc_t)


def kernel(x_cat, tables):
    tab_t = jnp.transpose(tables, (0, 2, 1))
    xc_t = jnp.transpose(x_cat.astype(jnp.int32))
    out_t = _run(tab_t, xc_t)
    return jnp.transpose(out_t)
